# baseline (speedup 1.0000x reference)
import functools

import jax
import jax.numpy as jnp
from jax.experimental import pallas as pl
from jax.experimental.pallas import tpu as pltpu

EPS = 1e-5
f32 = jnp.float32


def _shift_rows(p, d):
    if d == 0:
        return p
    if d == 1:
        return jnp.concatenate([p[1:, :], p[:1, :]], axis=0)
    return jnp.concatenate([p[-1:, :], p[:-1, :]], axis=0)


def _shift_lanes(p, d):
    if d == 0:
        return p
    if d == 1:
        return jnp.concatenate([p[:, 1:], p[:, :1]], axis=1)
    return jnp.concatenate([p[:, -1:], p[:, :-1]], axis=1)


def _build_planes(x_ref, b0, nin, h, w):
    zc = jnp.zeros((h, 1), f32)
    zr = jnp.zeros((1, 2 * (w + 2)), f32)
    planes = []
    for c in range(nin):
        row = jnp.concatenate(
            [zc, x_ref[b0, c], zc, zc, x_ref[b0 + 1, c], zc], axis=1)
        planes.append(jnp.concatenate([zr, row, zr], axis=0))
    return planes


def _depthwise(planes, dwb_ref, kpl):
    zs = []
    for c, p in enumerate(planes):
        rows = [_shift_rows(p, dh) for dh in (-1, 0, 1)]
        accs = [None] * kpl
        for kh in range(3):
            for kw in range(3):
                tap = _shift_lanes(rows[kh], kw - 1)
                j = kh * 3 + kw
                for k in range(kpl):
                    m = c * kpl + k
                    t = tap * dwb_ref[m * 9 + j:m * 9 + j + 1, :]
                    accs[k] = t if accs[k] is None else accs[k] + t
        zs.extend(accs)
    return zs


def _interior_mask(hp, l):
    wp = l // 2
    r = jax.lax.broadcasted_iota(jnp.int32, (hp, l), 0)
    q = jax.lax.broadcasted_iota(jnp.int32, (hp, l), 1)
    q = jax.lax.rem(q, wp)
    ok = (r >= 1) & (r <= hp - 2) & (q >= 1) & (q <= wp - 2)
    return jnp.where(ok, 1.0, 0.0).astype(f32)


def _psum(x, gs):
    if gs == 1:
        return jnp.sum(x, axis=0, keepdims=True)
    return jnp.sum(x.reshape(x.shape[0] // gs, gs, x.shape[1]), axis=0)


def _pointwise(acts, pwb_ref, nout):
    cmid = len(acts)
    outs = [None] * nout
    for m in range(cmid):
        for o in range(nout):
            t = acts[m] * pwb_ref[o * cmid + m:o * cmid + m + 1, :]
            outs[o] = t if outs[o] is None else outs[o] + t
    return [o.astype(f32) for o in outs]


def _acts_from_cache(z_ref, pair, a1_ref, b1_ref, maskb, cmid):
    bf16 = jnp.bfloat16
    acts = []
    for m in range(cmid):
        a = a1_ref[m:m + 1, :].astype(bf16)
        b = b1_ref[m:m + 1, :].astype(bf16)
        v = jnp.maximum(z_ref[pair, m] * a + b, jnp.array(0, bf16))
        acts.append(jnp.where(maskb, v, jnp.array(0, bf16)))
    return acts


def _bn_params(s_ref, ss_ref, g_ref, b_ref, cnt, gs):
    c, l = g_ref.shape
    s = jnp.sum(s_ref[...].reshape(c, gs, l), axis=(1, 2)).reshape(c, 1)
    ss = jnp.sum(ss_ref[...].reshape(c, gs, l), axis=(1, 2)).reshape(c, 1)
    mean = s * (1.0 / cnt)
    var = jnp.maximum(ss * (1.0 / cnt) - mean * mean, 0.0)
    a = g_ref[...] * jax.lax.rsqrt(jnp.broadcast_to(var, g_ref.shape) + EPS)
    b = b_ref[...] - jnp.broadcast_to(mean, g_ref.shape) * a
    return a, b


def _fused_kernel(x_ref, dwb_ref, pwb_ref, g1_ref, b1i_ref, g2_ref, b2i_ref,
                  out_ref,
                  z_ref, s1_ref, ss1_ref, a1_ref, b1_ref,
                  s2_ref, ss2_ref, a2_ref, b2_ref,
                  *, nseq, nb, nin, kpl, cmid, nout, h, w, n):
    i = pl.program_id(0)
    pairs = nb // 2
    hp, wp = h + 2, w + 2
    l = 2 * wp
    gs = 8 if hp % 8 == 0 else 1
    mask = _interior_mask(hp, l)
    maskb = mask > 0.5

    @pl.when(i == 0)
    def _():
        s1_ref[...] = jnp.zeros_like(s1_ref)
        ss1_ref[...] = jnp.zeros_like(ss1_ref)
        s2_ref[...] = jnp.zeros_like(s2_ref)
        ss2_ref[...] = jnp.zeros_like(ss2_ref)

    @pl.when(i < nseq)
    def _():
        srows, ssrows = None, None
        for p in range(pairs):
            zs = _depthwise(_build_planes(x_ref, 2 * p, nin, h, w),
                            dwb_ref, kpl)
            sr, sq = [], []
            for m, z in enumerate(zs):
                z_ref[i * pairs + p, m] = z.astype(jnp.bfloat16)
                zm = z * mask
                sr.append(_psum(zm, gs))
                sq.append(_psum(zm * zm, gs))
            srows = sr if srows is None else [a + b for a, b in zip(srows, sr)]
            ssrows = (sq if ssrows is None
                      else [a + b for a, b in zip(ssrows, sq)])
        s1_ref[...] += jnp.concatenate(srows, axis=0)
        ss1_ref[...] += jnp.concatenate(ssrows, axis=0)

    @pl.when(i == nseq - 1)
    def _():
        a, b = _bn_params(s1_ref, ss1_ref, g1_ref, b1i_ref,
                          float(n * h * w), gs)
        a1_ref[...] = a
        b1_ref[...] = b

    @pl.when((i >= nseq) & (i < 2 * nseq))
    def _():
        srows, ssrows = None, None
        for p in range(pairs):
            acts = _acts_from_cache(z_ref, (i - nseq) * pairs + p,
                                    a1_ref, b1_ref, maskb, cmid)
            pws = _pointwise(acts, pwb_ref, nout)
            sr, sq = [], []
            for pw in pws:
                sr.append(_psum(pw, gs))
                sq.append(_psum(pw * pw, gs))
            srows = sr if srows is None else [a + b for a, b in zip(srows, sr)]
            ssrows = (sq if ssrows is None
                      else [a + b for a, b in zip(ssrows, sq)])
        s2_ref[...] += jnp.concatenate(srows, axis=0)
        ss2_ref[...] += jnp.concatenate(ssrows, axis=0)

    @pl.when(i == 2 * nseq - 1)
    def _():
        a, b = _bn_params(s2_ref, ss2_ref, g2_ref, b2i_ref,
                          float(n * (h + 2) * (w + 2)), gs)
        a2_ref[...] = a
        b2_ref[...] = b

    @pl.when(i >= 2 * nseq)
    def _():
        for p in range(pairs):
            acts = _acts_from_cache(z_ref, (i - 2 * nseq) * pairs + p,
                                    a1_ref, b1_ref, maskb, cmid)
            pws = _pointwise(acts, pwb_ref, nout)
            for o, pw in enumerate(pws):
                val = jnp.maximum(
                    pw * a2_ref[o:o + 1, :] + b2_ref[o:o + 1, :], 0.0)
                out_ref[2 * p, o] = val[:, :wp]
                out_ref[2 * p + 1, o] = val[:, wp:]


def kernel(x, dw_w, pw_w, g1, b1, g2, b2):
    N, NIN, H, W = x.shape
    CMID = dw_w.shape[0]
    NOUT = pw_w.shape[0]
    KPL = CMID // NIN
    Hp, Wp = H + 2, W + 2
    L = 2 * Wp
    NB = 16 if N % 16 == 0 else (8 if N % 8 == 0 else 2)
    GS = 8 if Hp % 8 == 0 else 1
    NSEQ = N // NB

    dwb = jnp.broadcast_to(dw_w.astype(f32).reshape(CMID * 9, 1),
                           (CMID * 9, L))
    pwm = pw_w.astype(f32)[:, :, 0, 0]
    pwb = jnp.broadcast_to(pwm.reshape(NOUT * CMID, 1).astype(jnp.bfloat16),
                           (NOUT * CMID, L))
    g1b = jnp.broadcast_to(g1.astype(f32).reshape(CMID, 1), (CMID, L))
    b1b = jnp.broadcast_to(b1.astype(f32).reshape(CMID, 1), (CMID, L))
    g2b = jnp.broadcast_to(g2.astype(f32).reshape(NOUT, 1), (NOUT, L))
    b2b = jnp.broadcast_to(b2.astype(f32).reshape(NOUT, 1), (NOUT, L))

    last_a = NSEQ - 1
    x_spec = pl.BlockSpec(
        (NB, NIN, H, W),
        lambda i: (jnp.minimum(i, last_a), 0, 0, 0))

    def cspec(shape):
        nd = len(shape)
        return pl.BlockSpec(shape, lambda i, nd=nd: (0,) * nd)

    base_c = 2 * NSEQ
    out_spec = pl.BlockSpec(
        (NB, NOUT, Hp, Wp),
        lambda i: (jnp.maximum(i - base_c, 0), 0, 0, 0))

    out = pl.pallas_call(
        functools.partial(_fused_kernel, nseq=NSEQ, nb=NB, nin=NIN, kpl=KPL,
                          cmid=CMID, nout=NOUT, h=H, w=W, n=N),
        out_shape=jax.ShapeDtypeStruct((N, NOUT, Hp, Wp), f32),
        grid=(3 * NSEQ,),
        in_specs=[x_spec, cspec((CMID * 9, L)), cspec((NOUT * CMID, L)),
                  cspec((CMID, L)), cspec((CMID, L)),
                  cspec((NOUT, L)), cspec((NOUT, L))],
        out_specs=out_spec,
        scratch_shapes=[
            pltpu.VMEM((N // 2, CMID, Hp, L), jnp.bfloat16),
            pltpu.VMEM((CMID * GS, L), f32),
            pltpu.VMEM((CMID * GS, L), f32),
            pltpu.VMEM((CMID, L), f32), pltpu.VMEM((CMID, L), f32),
            pltpu.VMEM((NOUT * GS, L), f32),
            pltpu.VMEM((NOUT * GS, L), f32),
            pltpu.VMEM((NOUT, L), f32), pltpu.VMEM((NOUT, L), f32),
        ],
        compiler_params=pltpu.CompilerParams(
            dimension_semantics=("arbitrary",),
            vmem_limit_bytes=60 * 1024 * 1024),
    )(x, dwb, pwb, g1b, b1b, g2b, b2b)
    return out


# speedup vs baseline: 1.0084x; 1.0084x over previous
import functools

import jax
import jax.numpy as jnp
from jax.experimental import pallas as pl
from jax.experimental.pallas import tpu as pltpu

EPS = 1e-5
f32 = jnp.float32


def _shift_rows(p, d):
    if d == 0:
        return p
    if d == 1:
        return jnp.concatenate([p[1:, :], p[:1, :]], axis=0)
    return jnp.concatenate([p[-1:, :], p[:-1, :]], axis=0)


def _shift_lanes(p, d):
    if d == 0:
        return p
    if d == 1:
        return jnp.concatenate([p[:, 1:], p[:, :1]], axis=1)
    return jnp.concatenate([p[:, -1:], p[:, :-1]], axis=1)


def _build_planes(x_ref, b0, nin, h, w):
    zc = jnp.zeros((h, 1), f32)
    zr = jnp.zeros((1, 2 * (w + 2)), f32)
    planes = []
    for c in range(nin):
        row = jnp.concatenate(
            [zc, x_ref[b0, c], zc, zc, x_ref[b0 + 1, c], zc], axis=1)
        planes.append(jnp.concatenate([zr, row, zr], axis=0))
    return planes


def _depthwise(planes, dwb_ref, kpl):
    zs = []
    for c, p in enumerate(planes):
        rows = [_shift_rows(p, dh) for dh in (-1, 0, 1)]
        accs = [None] * kpl
        for kh in range(3):
            for kw in range(3):
                tap = _shift_lanes(rows[kh], kw - 1)
                j = kh * 3 + kw
                for k in range(kpl):
                    m = c * kpl + k
                    t = tap * dwb_ref[m * 9 + j:m * 9 + j + 1, :]
                    accs[k] = t if accs[k] is None else accs[k] + t
        zs.extend(accs)
    return zs


def _interior_mask(hp, l):
    wp = l // 2
    r = jax.lax.broadcasted_iota(jnp.int32, (hp, l), 0)
    q = jax.lax.broadcasted_iota(jnp.int32, (hp, l), 1)
    q = jax.lax.rem(q, wp)
    ok = (r >= 1) & (r <= hp - 2) & (q >= 1) & (q <= wp - 2)
    return jnp.where(ok, 1.0, 0.0).astype(f32)


def _pointwise(acts, pwb_ref, nout):
    cmid = len(acts)
    outs = [None] * nout
    for m in range(cmid):
        for o in range(nout):
            t = acts[m] * pwb_ref[o * cmid + m:o * cmid + m + 1, :]
            outs[o] = t if outs[o] is None else outs[o] + t
    return outs


def _stats1_kernel(x_ref, dwb_ref, s_ref, ss_ref, z_ref,
                   *, nb, nin, kpl, h, w):
    @pl.when(pl.program_id(1) == 0)
    def _():
        s_ref[...] = jnp.zeros_like(s_ref)
        ss_ref[...] = jnp.zeros_like(ss_ref)

    mask = _interior_mask(h + 2, 2 * (w + 2))
    srows, ssrows = None, None
    for p in range(nb // 2):
        zs = _depthwise(_build_planes(x_ref, 2 * p, nin, h, w), dwb_ref, kpl)
        sr = []
        sq = []
        for m, z in enumerate(zs):
            z_ref[p, m] = z.astype(jnp.bfloat16)
            zm = z * mask
            sr.append(jnp.sum(zm, axis=0, keepdims=True))
            sq.append(jnp.sum(zm * zm, axis=0, keepdims=True))
        srows = sr if srows is None else [a + b for a, b in zip(srows, sr)]
        ssrows = sq if ssrows is None else [a + b for a, b in zip(ssrows, sq)]
    s_ref[0] += jnp.concatenate(srows, axis=0)
    ss_ref[0] += jnp.concatenate(ssrows, axis=0)


def _stats2_kernel(z_ref, a1_ref, b1_ref, pwb_ref, s_ref, ss_ref,
                   *, nb, cmid, nout, h, w):
    @pl.when(pl.program_id(1) == 0)
    def _():
        s_ref[...] = jnp.zeros_like(s_ref)
        ss_ref[...] = jnp.zeros_like(ss_ref)

    mask = _interior_mask(h + 2, 2 * (w + 2))
    srows, ssrows = None, None
    for p in range(nb // 2):
        zs = [z_ref[p, m].astype(f32) for m in range(cmid)]
        acts = [jnp.maximum(z * a1_ref[m:m + 1, :] + b1_ref[m:m + 1, :], 0.0)
                * mask for m, z in enumerate(zs)]
        pws = _pointwise(acts, pwb_ref, nout)
        sr = []
        sq = []
        for pw in pws:
            sr.append(jnp.sum(pw, axis=0, keepdims=True))
            sq.append(jnp.sum(pw * pw, axis=0, keepdims=True))
        srows = sr if srows is None else [a + b for a, b in zip(srows, sr)]
        ssrows = sq if ssrows is None else [a + b for a, b in zip(ssrows, sq)]
    s_ref[0] += jnp.concatenate(srows, axis=0)
    ss_ref[0] += jnp.concatenate(ssrows, axis=0)


def _final_kernel(z_ref, a1_ref, b1_ref, pwb_ref, a2_ref, b2_ref,
                  out_ref, *, nb, cmid, nout, h, w):
    mask = _interior_mask(h + 2, 2 * (w + 2))
    wp = w + 2
    for p in range(nb // 2):
        zs = [z_ref[p, m].astype(f32) for m in range(cmid)]
        acts = [jnp.maximum(z * a1_ref[m:m + 1, :] + b1_ref[m:m + 1, :], 0.0)
                * mask for m, z in enumerate(zs)]
        pws = _pointwise(acts, pwb_ref, nout)
        for o, pw in enumerate(pws):
            val = jnp.maximum(pw * a2_ref[o:o + 1, :] + b2_ref[o:o + 1, :],
                              0.0)
            out_ref[2 * p, o] = val[:, :wp]
            out_ref[2 * p + 1, o] = val[:, wp:]


def kernel(x, dw_w, pw_w, g1, b1, g2, b2):
    N, NIN, H, W = x.shape
    CMID = dw_w.shape[0]
    NOUT = pw_w.shape[0]
    KPL = CMID // NIN
    Hp, Wp = H + 2, W + 2
    L = 2 * Wp
    NB = 64 if N % 64 == 0 else (8 if N % 8 == 0 else 2)
    NBLK = N // NB
    NCORE = 1
    NSEQ = NBLK // NCORE

    dwb = jnp.broadcast_to(dw_w.astype(f32).reshape(CMID * 9, 1), (CMID * 9, L))
    pwm = pw_w.astype(f32)[:, :, 0, 0]
    pwb = jnp.broadcast_to(pwm.reshape(NOUT * CMID, 1), (NOUT * CMID, L))

    x_spec = pl.BlockSpec((NB, NIN, H, W),
                          lambda k, n: (k * NSEQ + n, 0, 0, 0))

    def cspec(shape):
        nd = len(shape)
        return pl.BlockSpec(shape, lambda k, n, nd=nd: (0,) * nd)

    def accspec(shape):
        return pl.BlockSpec((1,) + shape,
                            lambda k, n: (k,) + (0,) * len(shape))

    cp = pltpu.CompilerParams(
        dimension_semantics=("arbitrary", "arbitrary"),
        vmem_limit_bytes=48 * 1024 * 1024)

    z_spec = pl.BlockSpec((NB // 2, CMID, Hp, L),
                          lambda k, n: (k * NSEQ + n, 0, 0, 0))
    s1, ss1, zc = pl.pallas_call(
        functools.partial(_stats1_kernel, nb=NB, nin=NIN, kpl=KPL, h=H, w=W),
        out_shape=(jax.ShapeDtypeStruct((NCORE, CMID, L), f32),
                   jax.ShapeDtypeStruct((NCORE, CMID, L), f32),
                   jax.ShapeDtypeStruct((N // 2, CMID, Hp, L), jnp.bfloat16)),
        grid=(NCORE, NSEQ),
        in_specs=[x_spec, cspec((CMID * 9, L))],
        out_specs=(accspec((CMID, L)), accspec((CMID, L)), z_spec),
        compiler_params=cp,
    )(x, dwb)

    cnt1 = float(N * H * W)
    mean1 = jnp.sum(s1, axis=(0, 2)) / cnt1
    var1 = jnp.maximum(jnp.sum(ss1, axis=(0, 2)) / cnt1 - mean1 * mean1, 0.0)
    a1 = g1.astype(f32) * jax.lax.rsqrt(var1 + EPS)
    b1v = b1.astype(f32) - mean1 * a1
    a1b = jnp.broadcast_to(a1.reshape(CMID, 1), (CMID, L))
    b1b = jnp.broadcast_to(b1v.reshape(CMID, 1), (CMID, L))

    s2, ss2 = pl.pallas_call(
        functools.partial(_stats2_kernel, nb=NB, cmid=CMID,
                          nout=NOUT, h=H, w=W),
        out_shape=(jax.ShapeDtypeStruct((NCORE, NOUT, L), f32),
                   jax.ShapeDtypeStruct((NCORE, NOUT, L), f32)),
        grid=(NCORE, NSEQ),
        in_specs=[z_spec, cspec((CMID, L)),
                  cspec((CMID, L)), cspec((NOUT * CMID, L))],
        out_specs=(accspec((NOUT, L)), accspec((NOUT, L))),
        compiler_params=cp,
    )(zc, a1b, b1b, pwb)

    cnt2 = float(N * Hp * Wp)
    mean2 = jnp.sum(s2, axis=(0, 2)) / cnt2
    var2 = jnp.maximum(jnp.sum(ss2, axis=(0, 2)) / cnt2 - mean2 * mean2, 0.0)
    a2 = g2.astype(f32) * jax.lax.rsqrt(var2 + EPS)
    b2v = b2.astype(f32) - mean2 * a2
    a2b = jnp.broadcast_to(a2.reshape(NOUT, 1), (NOUT, L))
    b2b = jnp.broadcast_to(b2v.reshape(NOUT, 1), (NOUT, L))

    out = pl.pallas_call(
        functools.partial(_final_kernel, nb=NB, cmid=CMID,
                          nout=NOUT, h=H, w=W),
        out_shape=jax.ShapeDtypeStruct((N, NOUT, Hp, Wp), f32),
        grid=(NCORE, NSEQ),
        in_specs=[z_spec, cspec((CMID, L)),
                  cspec((CMID, L)), cspec((NOUT * CMID, L)),
                  cspec((NOUT, L)), cspec((NOUT, L))],
        out_specs=pl.BlockSpec((NB, NOUT, Hp, Wp),
                               lambda k, n: (k * NSEQ + n, 0, 0, 0)),
        compiler_params=cp,
    )(zc, a1b, b1b, pwb, a2b, b2b)
    return out


# speedup vs baseline: 1.0091x; 1.0007x over previous
import functools

import jax
import jax.numpy as jnp
from jax.experimental import pallas as pl
from jax.experimental.pallas import tpu as pltpu

EPS = 1e-5
f32 = jnp.float32


def _shift_rows(p, d):
    if d == 0:
        return p
    if d == 1:
        return jnp.concatenate([p[1:, :], p[:1, :]], axis=0)
    return jnp.concatenate([p[-1:, :], p[:-1, :]], axis=0)


def _shift_lanes(p, d):
    if d == 0:
        return p
    if d == 1:
        return jnp.concatenate([p[:, 1:], p[:, :1]], axis=1)
    return jnp.concatenate([p[:, -1:], p[:, :-1]], axis=1)


def _build_planes(x_ref, b0, nin, h, w):
    zc = jnp.zeros((h, 1), f32)
    zr = jnp.zeros((1, 2 * (w + 2)), f32)
    planes = []
    for c in range(nin):
        row = jnp.concatenate(
            [zc, x_ref[b0, c], zc, zc, x_ref[b0 + 1, c], zc], axis=1)
        planes.append(jnp.concatenate([zr, row, zr], axis=0))
    return planes


def _depthwise(planes, dwb_ref, kpl):
    zs = []
    for c, p in enumerate(planes):
        rows = [_shift_rows(p, dh) for dh in (-1, 0, 1)]
        accs = [None] * kpl
        for kh in range(3):
            for kw in range(3):
                tap = _shift_lanes(rows[kh], kw - 1)
                j = kh * 3 + kw
                for k in range(kpl):
                    m = c * kpl + k
                    t = tap * dwb_ref[m * 9 + j:m * 9 + j + 1, :]
                    accs[k] = t if accs[k] is None else accs[k] + t
        zs.extend(accs)
    return zs


def _interior_mask(hp, l):
    wp = l // 2
    r = jax.lax.broadcasted_iota(jnp.int32, (hp, l), 0)
    q = jax.lax.broadcasted_iota(jnp.int32, (hp, l), 1)
    q = jax.lax.rem(q, wp)
    ok = (r >= 1) & (r <= hp - 2) & (q >= 1) & (q <= wp - 2)
    return jnp.where(ok, 1.0, 0.0).astype(f32)


def _pointwise(acts, pwb_ref, nout):
    cmid = len(acts)
    outs = [None] * nout
    for m in range(cmid):
        for o in range(nout):
            t = acts[m] * pwb_ref[o * cmid + m:o * cmid + m + 1, :]
            outs[o] = t if outs[o] is None else outs[o] + t
    return outs


def _stats1_kernel(x_ref, dwb_ref, s_ref, ss_ref, z_ref,
                   *, nb, nin, kpl, h, w):
    @pl.when(pl.program_id(1) == 0)
    def _():
        s_ref[...] = jnp.zeros_like(s_ref)
        ss_ref[...] = jnp.zeros_like(ss_ref)

    mask = _interior_mask(h + 2, 2 * (w + 2))
    srows, ssrows = None, None
    for p in range(nb // 2):
        zs = _depthwise(_build_planes(x_ref, 2 * p, nin, h, w), dwb_ref, kpl)
        sr = []
        sq = []
        for m, z in enumerate(zs):
            z_ref[p, m] = z.astype(jnp.bfloat16)
            zm = z * mask
            sr.append(jnp.sum(zm, axis=0, keepdims=True))
            sq.append(jnp.sum(zm * zm, axis=0, keepdims=True))
        srows = sr if srows is None else [a + b for a, b in zip(srows, sr)]
        ssrows = sq if ssrows is None else [a + b for a, b in zip(ssrows, sq)]
    s_ref[0] += jnp.concatenate(srows, axis=0)
    ss_ref[0] += jnp.concatenate(ssrows, axis=0)


def _stats2_kernel(z_ref, a1_ref, b1_ref, pwb_ref, s_ref, ss_ref,
                   *, nb, cmid, nout, h, w):
    @pl.when(pl.program_id(1) == 0)
    def _():
        s_ref[...] = jnp.zeros_like(s_ref)
        ss_ref[...] = jnp.zeros_like(ss_ref)

    mask = _interior_mask(h + 2, 2 * (w + 2))
    srows, ssrows = None, None
    for p in range(nb // 2):
        zs = [z_ref[p, m].astype(f32) for m in range(cmid)]
        acts = [jnp.maximum(z * a1_ref[m:m + 1, :] + b1_ref[m:m + 1, :], 0.0)
                * mask for m, z in enumerate(zs)]
        pws = _pointwise(acts, pwb_ref, nout)
        sr = []
        sq = []
        for pw in pws:
            sr.append(jnp.sum(pw, axis=0, keepdims=True))
            sq.append(jnp.sum(pw * pw, axis=0, keepdims=True))
        srows = sr if srows is None else [a + b for a, b in zip(srows, sr)]
        ssrows = sq if ssrows is None else [a + b for a, b in zip(ssrows, sq)]
    s_ref[0] += jnp.concatenate(srows, axis=0)
    ss_ref[0] += jnp.concatenate(ssrows, axis=0)


def _final_kernel(z_ref, a1_ref, b1_ref, pwb_ref, a2_ref, b2_ref,
                  out_ref, *, nb, cmid, nout, h, w):
    mask = _interior_mask(h + 2, 2 * (w + 2))
    wp = w + 2
    for p in range(nb // 2):
        zs = [z_ref[p, m].astype(f32) for m in range(cmid)]
        acts = [jnp.maximum(z * a1_ref[m:m + 1, :] + b1_ref[m:m + 1, :], 0.0)
                * mask for m, z in enumerate(zs)]
        pws = _pointwise(acts, pwb_ref, nout)
        for o, pw in enumerate(pws):
            val = jnp.maximum(pw * a2_ref[o:o + 1, :] + b2_ref[o:o + 1, :],
                              0.0)
            out_ref[2 * p, o] = val[:, :wp]
            out_ref[2 * p + 1, o] = val[:, wp:]


def kernel(x, dw_w, pw_w, g1, b1, g2, b2):
    N, NIN, H, W = x.shape
    CMID = dw_w.shape[0]
    NOUT = pw_w.shape[0]
    KPL = CMID // NIN
    Hp, Wp = H + 2, W + 2
    L = 2 * Wp
    NB = 32 if N % 32 == 0 else (8 if N % 8 == 0 else 2)
    NBLK = N // NB
    NCORE = 1
    NSEQ = NBLK // NCORE

    dwb = jnp.broadcast_to(dw_w.astype(f32).reshape(CMID * 9, 1), (CMID * 9, L))
    pwm = pw_w.astype(f32)[:, :, 0, 0]
    pwb = jnp.broadcast_to(pwm.reshape(NOUT * CMID, 1), (NOUT * CMID, L))

    x_spec = pl.BlockSpec((NB, NIN, H, W),
                          lambda k, n: (k * NSEQ + n, 0, 0, 0))

    def cspec(shape):
        nd = len(shape)
        return pl.BlockSpec(shape, lambda k, n, nd=nd: (0,) * nd)

    def accspec(shape):
        return pl.BlockSpec((1,) + shape,
                            lambda k, n: (k,) + (0,) * len(shape))

    cp = pltpu.CompilerParams(
        dimension_semantics=("arbitrary", "arbitrary"),
        vmem_limit_bytes=48 * 1024 * 1024)

    z_spec = pl.BlockSpec((NB // 2, CMID, Hp, L),
                          lambda k, n: (k * NSEQ + n, 0, 0, 0))
    s1, ss1, zc = pl.pallas_call(
        functools.partial(_stats1_kernel, nb=NB, nin=NIN, kpl=KPL, h=H, w=W),
        out_shape=(jax.ShapeDtypeStruct((NCORE, CMID, L), f32),
                   jax.ShapeDtypeStruct((NCORE, CMID, L), f32),
                   jax.ShapeDtypeStruct((N // 2, CMID, Hp, L), jnp.bfloat16)),
        grid=(NCORE, NSEQ),
        in_specs=[x_spec, cspec((CMID * 9, L))],
        out_specs=(accspec((CMID, L)), accspec((CMID, L)), z_spec),
        compiler_params=cp,
    )(x, dwb)

    cnt1 = float(N * H * W)
    mean1 = jnp.sum(s1, axis=(0, 2)) / cnt1
    var1 = jnp.maximum(jnp.sum(ss1, axis=(0, 2)) / cnt1 - mean1 * mean1, 0.0)
    a1 = g1.astype(f32) * jax.lax.rsqrt(var1 + EPS)
    b1v = b1.astype(f32) - mean1 * a1
    a1b = jnp.broadcast_to(a1.reshape(CMID, 1), (CMID, L))
    b1b = jnp.broadcast_to(b1v.reshape(CMID, 1), (CMID, L))

    s2, ss2 = pl.pallas_call(
        functools.partial(_stats2_kernel, nb=NB, cmid=CMID,
                          nout=NOUT, h=H, w=W),
        out_shape=(jax.ShapeDtypeStruct((NCORE, NOUT, L), f32),
                   jax.ShapeDtypeStruct((NCORE, NOUT, L), f32)),
        grid=(NCORE, NSEQ),
        in_specs=[z_spec, cspec((CMID, L)),
                  cspec((CMID, L)), cspec((NOUT * CMID, L))],
        out_specs=(accspec((NOUT, L)), accspec((NOUT, L))),
        compiler_params=cp,
    )(zc, a1b, b1b, pwb)

    cnt2 = float(N * Hp * Wp)
    mean2 = jnp.sum(s2, axis=(0, 2)) / cnt2
    var2 = jnp.maximum(jnp.sum(ss2, axis=(0, 2)) / cnt2 - mean2 * mean2, 0.0)
    a2 = g2.astype(f32) * jax.lax.rsqrt(var2 + EPS)
    b2v = b2.astype(f32) - mean2 * a2
    a2b = jnp.broadcast_to(a2.reshape(NOUT, 1), (NOUT, L))
    b2b = jnp.broadcast_to(b2v.reshape(NOUT, 1), (NOUT, L))

    out = pl.pallas_call(
        functools.partial(_final_kernel, nb=NB, cmid=CMID,
                          nout=NOUT, h=H, w=W),
        out_shape=jax.ShapeDtypeStruct((N, NOUT, Hp, Wp), f32),
        grid=(NCORE, NSEQ),
        in_specs=[z_spec, cspec((CMID, L)),
                  cspec((CMID, L)), cspec((NOUT * CMID, L)),
                  cspec((NOUT, L)), cspec((NOUT, L))],
        out_specs=pl.BlockSpec((NB, NOUT, Hp, Wp),
                               lambda k, n: (k * NSEQ + n, 0, 0, 0)),
        compiler_params=cp,
    )(zc, a1b, b1b, pwb, a2b, b2b)
    return out
